# baseline (speedup 1.0000x reference)
import jax
import jax.numpy as jnp
from jax import lax
from jax.experimental import pallas as pl
from jax.experimental.pallas import tpu as pltpu
from jax.experimental.pallas import tpu_sc as plsc

NC = 2
NS = 16
NW = NC * NS
CH = 128
K = 4


def _make_gather(n_ids: int, d: int, interpret: bool = False):
    assert n_ids % (NW * CH * K * 2) == 0
    cpw = n_ids // (NW * CH)
    ng = cpw // K
    b_per_w = cpw * CH
    mesh = plsc.VectorSubcoreMesh(
        core_axis_name="c", subcore_axis_name="s", num_cores=NC, num_subcores=NS
    )

    def body(idx_hbm, table_hbm, out_hbm, idx_v, rows0, rows1,
             gsem0, gsem1, osem0, osem1):
        rows = (rows0, rows1)
        gsem = (gsem0, gsem1)
        osem = (osem0, osem1)
        wid = lax.axis_index("s") * NC + lax.axis_index("c")
        wbase = wid * b_per_w
        pltpu.sync_copy(idx_hbm.at[wid], idx_v)

        def fire_group(g, p):
            for k in range(K):
                pltpu.async_copy(
                    table_hbm.at[idx_v.at[g * K + k]],
                    rows[p].at[pl.ds(k * CH, CH)],
                    gsem[p],
                )

        def drain_group(p):
            for k in range(K):
                pltpu.make_async_copy(
                    out_hbm.at[pl.ds(0, CH)],
                    rows[p].at[pl.ds(k * CH, CH)],
                    gsem[p],
                ).wait()

        fire_group(0, 0)

        @pl.loop(0, ng // 2)
        def _(gl):
            for p in range(2):
                g = gl * 2 + p
                q = 1 - p
                @pl.when(g >= 1)
                def _():
                    pltpu.make_async_copy(
                        out_hbm.at[pl.ds(0, K * CH)], rows[q], osem[q]
                    ).wait()

                @pl.when(g + 1 < ng)
                def _():
                    fire_group(g + 1, q)

                drain_group(p)
                pltpu.async_copy(
                    rows[p], out_hbm.at[pl.ds(wbase + g * K * CH, K * CH)], osem[p]
                )

        lp = (ng - 1) % 2
        pltpu.make_async_copy(
            out_hbm.at[pl.ds(0, K * CH)], rows[lp], osem[lp]
        ).wait()

    return pl.kernel(
        body,
        out_type=jax.ShapeDtypeStruct((n_ids, d), jnp.float32),
        mesh=mesh,
        scratch_types=(
            pltpu.VMEM((cpw, CH), jnp.int32),
            pltpu.VMEM((K * CH, d), jnp.float32),
            pltpu.VMEM((K * CH, d), jnp.float32),
            pltpu.SemaphoreType.DMA,
            pltpu.SemaphoreType.DMA,
            pltpu.SemaphoreType.DMA,
            pltpu.SemaphoreType.DMA,
        ),
        compiler_params=pltpu.CompilerParams(use_tc_tiling_on_sc=False),
        interpret=interpret,
    )


def kernel(token_ids, embedding_matrix):
    b, t = token_ids.shape
    n = b * t
    d = embedding_matrix.shape[1]
    idx = token_ids.astype(jnp.int32).reshape(NW, n // (NW * CH), CH)
    out = _make_gather(n, d)(idx, embedding_matrix)
    return out.reshape(b, t, d)
